# baseline (speedup 1.0000x reference)
import jax
import jax.numpy as jnp
from jax import lax
from jax.experimental import pallas as pl
from jax.experimental.pallas import tpu as pltpu
from jax.experimental.pallas import tpu_sc as plsc

NC, NS, L = 2, 16, 16
NW = NC * NS
B, T, H = 1024, 50, 160
N = B * T
BB = 128
NBT = B // BB
UNITS = T * NBT
UPW = 13
GP = BB // L
HT, HI = H // 8, 8


def _body(idx_t, reg_t, e0, e1, e2, e3, ws, bs, wo, bo, wu, bu, wk, bk,
          unk_o, kno_o, obs_o, st_o,
          idx_v, reg_v, rwa, rwb, rb0, rb1,
          ws_m, bs_m, wo_m, bo_m, wu_m, bu_m, wk_m, bk_m,
          gsa, gsb, osem0, osem1, isem0, isem1):
    wid = lax.axis_index("s") * NC + lax.axis_index("c")
    lanes = lax.iota(jnp.int32, L)
    gsem = (gsa, gsb)
    isem = (isem0, isem1)
    osem = (osem0, osem1)
    rbufs = (rb0, rb1)
    rrows = (rwa, rwb)
    tabs = (e0, e1, e2, e3)
    bvecs = [g * L + lanes for g in range(GP)]

    pltpu.sync_copy(ws, ws_m)
    pltpu.sync_copy(bs, bs_m)
    pltpu.sync_copy(wo, wo_m)
    pltpu.sync_copy(bo, bo_m)
    pltpu.sync_copy(wu, wu_m)
    pltpu.sync_copy(bu, bu_m)
    pltpu.sync_copy(wk, wk_m)
    pltpu.sync_copy(bk, bk_m)

    def unit_tb(u):
        gid = u * NW + wid
        t = jnp.where(u == UPW, 0, gid // NBT)
        bt = jnp.where(u == UPW, lax.rem(wid, NBT), lax.rem(gid, NBT))
        return t, bt

    def fetch_io(u, d):
        t, bt = unit_tb(u)
        off = t * B + bt * BB
        pltpu.async_copy(idx_t.at[:, pl.ds(off, BB)], idx_v.at[d], isem[d])
        pltpu.async_copy(reg_t.at[:, pl.ds(off, BB)], reg_v.at[d], isem[d])

    def io_wait(d):
        pltpu.make_async_copy(idx_t.at[:, pl.ds(0, BB)], idx_v.at[d],
                              isem[d]).wait()
        pltpu.make_async_copy(reg_t.at[:, pl.ds(0, BB)], reg_v.at[d],
                              isem[d]).wait()

    def fire(d, k):
        pltpu.async_copy(tabs[k].at[idx_v.at[d, k]], rrows[k % 2],
                         gsem[k % 2])

    def gather_wait(d, k):
        pltpu.make_async_copy(tabs[k].at[idx_v.at[d, k]], rrows[k % 2],
                              gsem[k % 2]).wait()

    def ring_wait(j):
        pltpu.make_async_copy(
            rbufs[j % 2], kno_o.at[0, 0, :, 0, :, :], osem[j % 2]).wait()

    def lin_plane(d, rcol, w_m, b_m, j, dst):
        svs = [reg_v[d, rcol, pl.ds(g * L, L)] for g in range(GP)]
        rb = rbufs[j % 2]

        @plsc.parallel_loop(0, H, unroll=8)
        def _(h):
            hv = jnp.full((L,), h, jnp.int32)
            wh = plsc.load_gather(w_m, [hv])
            bh = plsc.load_gather(b_m, [hv])
            ht = h // 8
            hi = lax.rem(h, 8)
            for g in range(GP):
                rb[ht, hi, pl.ds(g * L, L)] = svs[g] * wh + bh

        pltpu.async_copy(rb, dst, osem[j % 2])

    def emb_plane(k, j, dst):
        src = rrows[k % 2]
        rb = rbufs[j % 2]

        @plsc.parallel_loop(0, H, unroll=8)
        def _(h):
            hv = jnp.full((L,), h, jnp.int32)
            ht = h // 8
            hi = lax.rem(h, 8)
            for g in range(GP):
                v = plsc.load_gather(src, [bvecs[g], hv])
                rb[ht, hi, pl.ds(g * L, L)] = v

        pltpu.async_copy(rb, dst, osem[j % 2])

    fetch_io(0, 0)

    def pair(p, carry):
        for d in range(2):
            u = 2 * p + d
            t, bt = unit_tb(u)
            gid = u * NW + wid

            io_wait(d)

            @pl.when(u + 1 <= UPW)
            def _():
                fetch_io(u + 1, 1 - d)

            @pl.when(jnp.logical_and(u < UPW, gid < UNITS))
            def _():
                fire(d, 0)
                fire(d, 1)

                @pl.when(u >= 1)
                def _():
                    ring_wait(0)

                lin_plane(d, 3, wu_m, bu_m, 0, unk_o.at[t, :, :, bt, 0, :])

                @pl.when(u >= 1)
                def _():
                    ring_wait(1)

                lin_plane(d, 1, wk_m, bk_m, 1, kno_o.at[t, 0, :, bt, :, :])
                ring_wait(2)
                lin_plane(d, 2, wk_m, bk_m, 2, kno_o.at[t, 1, :, bt, :, :])
                ring_wait(3)
                lin_plane(d, 0, wo_m, bo_m, 3, obs_o.at[t, :, :, bt, :])

                gather_wait(d, 0)
                ring_wait(4)
                emb_plane(0, 4, unk_o.at[t, :, :, bt, 1, :])
                fire(d, 2)
                gather_wait(d, 1)
                ring_wait(5)
                emb_plane(1, 5, kno_o.at[t, 2, :, bt, :, :])
                fire(d, 3)
                gather_wait(d, 2)
                ring_wait(6)
                emb_plane(2, 6, kno_o.at[t, 3, :, bt, :, :])
                gather_wait(d, 3)
                ring_wait(7)
                emb_plane(3, 7, kno_o.at[t, 4, :, bt, :, :])

            @pl.when(u == UPW)
            def _():
                fire(d, 0)
                gather_wait(d, 0)

                @pl.when(wid < NBT)
                def _():
                    sbt = lax.rem(wid, NBT)
                    ring_wait(0)
                    lin_plane(d, 3, ws_m, bs_m, 0, st_o.at[0, :, sbt, :, :])
                    ring_wait(1)
                    emb_plane(0, 1, st_o.at[1, :, sbt, :, :])
        return carry

    lax.fori_loop(0, (UPW + 2) // 2, pair, 0)
    ring_wait(0)
    ring_wait(1)


@jax.jit
def _run(idx_t, reg_t, e0, e1, e2, e3, ws, bs, wo, bo, wu, bu, wk, bk):
    mesh = plsc.VectorSubcoreMesh(core_axis_name="c", subcore_axis_name="s")
    f = pl.kernel(
        _body, mesh=mesh,
        compiler_params=pltpu.CompilerParams(
            needs_layout_passes=False, use_tc_tiling_on_sc=False),
        out_type=[
            jax.ShapeDtypeStruct((T, HT, HI, NBT, 2, BB), jnp.float32),
            jax.ShapeDtypeStruct((T, 5, HT, NBT, HI, BB), jnp.float32),
            jax.ShapeDtypeStruct((T, HT, HI, NBT, BB), jnp.float32),
            jax.ShapeDtypeStruct((2, HT, NBT, HI, BB), jnp.float32),
        ],
        scratch_types=(
            [pltpu.VMEM((2, 4, BB), jnp.int32),
             pltpu.VMEM((2, 4, BB), jnp.float32),
             pltpu.VMEM((BB, H), jnp.float32),
             pltpu.VMEM((BB, H), jnp.float32),
             pltpu.VMEM((HT, HI, BB), jnp.float32),
             pltpu.VMEM((HT, HI, BB), jnp.float32)]
            + [pltpu.VMEM((H,), jnp.float32)] * 8
            + [pltpu.SemaphoreType.DMA] * 6
        ),
    )
    return f(idx_t, reg_t, e0, e1, e2, e3, ws, bs, wo, bo, wu, bu, wk, bk)


def kernel(all_inputs, emb_0, emb_1, emb_2, emb_3, W_static, b_static,
           W_obs, b_obs, W_unknown, b_unknown, W_known, b_known):
    ai_t = all_inputs.transpose(2, 1, 0)
    idx_t = ai_t[4:].astype(jnp.int32).reshape(4, T * B)
    reg_t = ai_t[:4].reshape(4, T * B)
    unk_p, kno_p, obs_p, st_p = _run(
        idx_t, reg_t, emb_0, emb_1, emb_2, emb_3,
        W_static.reshape(H), b_static, W_obs.reshape(H), b_obs,
        W_unknown.reshape(H), b_unknown, W_known.reshape(H), b_known)
    unk = unk_p.transpose(3, 5, 0, 1, 2, 4).reshape(B, T, H, 2)
    kno = kno_p.transpose(3, 5, 0, 2, 4, 1).reshape(B, T, H, 5)
    obs = obs_p.transpose(3, 4, 0, 1, 2).reshape(B, T, H, 1)
    st = st_p.transpose(2, 4, 0, 1, 3).reshape(B, 2, H)
    return (unk, kno, obs, st)


# speedup vs baseline: 1.0235x; 1.0235x over previous
import jax
import jax.numpy as jnp
from jax import lax
from jax.experimental import pallas as pl
from jax.experimental.pallas import tpu as pltpu
from jax.experimental.pallas import tpu_sc as plsc

NC, NS, L = 2, 16, 16
NW = NC * NS
B, T, H = 1024, 50, 160
N = B * T
BB = 128
NBT = B // BB
UNITS = T * NBT
UPW = 13
GP = BB // L
HT, HI = H // 8, 8


def _body(idx_t, reg_t, e0, e1, e2, e3, ws, bs, wo, bo, wu, bu, wk, bk,
          unk_o, kno_o, obs_o, st_o,
          idx_v, reg_v, rwa, rwb, rb0, rb1, ubuf,
          ws_m, bs_m, wo_m, bo_m, wu_m, bu_m, wk_m, bk_m,
          gsa, gsb, osem0, osem1, isem0, isem1, usem):
    wid = lax.axis_index("s") * NC + lax.axis_index("c")
    lanes = lax.iota(jnp.int32, L)
    gsem = (gsa, gsb)
    isem = (isem0, isem1)
    osem = (osem0, osem1)
    rbufs = (rb0, rb1)
    rrows = (rwa, rwb)
    tabs = (e0, e1, e2, e3)
    bvecs = [g * L + lanes for g in range(GP)]

    pltpu.sync_copy(ws, ws_m)
    pltpu.sync_copy(bs, bs_m)
    pltpu.sync_copy(wo, wo_m)
    pltpu.sync_copy(bo, bo_m)
    pltpu.sync_copy(wu, wu_m)
    pltpu.sync_copy(bu, bu_m)
    pltpu.sync_copy(wk, wk_m)
    pltpu.sync_copy(bk, bk_m)

    def unit_tb(u):
        gid = u * NW + wid
        t = jnp.where(u == UPW, 0, gid // NBT)
        bt = jnp.where(u == UPW, lax.rem(wid, NBT), lax.rem(gid, NBT))
        return t, bt

    def fetch_io(u, d):
        t, bt = unit_tb(u)
        off = t * B + bt * BB
        pltpu.async_copy(idx_t.at[:, pl.ds(off, BB)], idx_v.at[d], isem[d])
        pltpu.async_copy(reg_t.at[:, pl.ds(off, BB)], reg_v.at[d], isem[d])

    def io_wait(d):
        pltpu.make_async_copy(idx_t.at[:, pl.ds(0, BB)], idx_v.at[d],
                              isem[d]).wait()
        pltpu.make_async_copy(reg_t.at[:, pl.ds(0, BB)], reg_v.at[d],
                              isem[d]).wait()

    def fire(d, k):
        pltpu.async_copy(tabs[k].at[idx_v.at[d, k]], rrows[k % 2],
                         gsem[k % 2])

    def gather_wait(d, k):
        pltpu.make_async_copy(tabs[k].at[idx_v.at[d, k]], rrows[k % 2],
                              gsem[k % 2]).wait()

    def ring_wait(j):
        pltpu.make_async_copy(
            rbufs[j % 2], kno_o.at[0, 0, :, 0, :, :], osem[j % 2]).wait()

    def uwait():
        pltpu.make_async_copy(ubuf, unk_o.at[0, :, :, 0, :, :], usem).wait()

    def lin_plane(d, rcol, w_m, b_m, j, dst):
        svs = [reg_v[d, rcol, pl.ds(g * L, L)] for g in range(GP)]
        rb = rbufs[j % 2]

        @plsc.parallel_loop(0, H, unroll=4)
        def _(h):
            hv = jnp.full((L,), h, jnp.int32)
            wh = plsc.load_gather(w_m, [hv])
            bh = plsc.load_gather(b_m, [hv])
            ht = h // 8
            hi = lax.rem(h, 8)
            for g in range(GP):
                rb[ht, hi, pl.ds(g * L, L)] = svs[g] * wh + bh

        pltpu.async_copy(rb, dst, osem[j % 2])

    def emb_plane(k, j, dst):
        src = rrows[k % 2]
        rb = rbufs[j % 2]

        @plsc.parallel_loop(0, H, unroll=4)
        def _(h):
            hv = jnp.full((L,), h, jnp.int32)
            ht = h // 8
            hi = lax.rem(h, 8)
            for g in range(GP):
                v = plsc.load_gather(src, [bvecs[g], hv])
                rb[ht, hi, pl.ds(g * L, L)] = v

        pltpu.async_copy(rb, dst, osem[j % 2])

    def ulin_build(d):
        svs = [reg_v[d, 3, pl.ds(g * L, L)] for g in range(GP)]

        @plsc.parallel_loop(0, H, unroll=4)
        def _(h):
            hv = jnp.full((L,), h, jnp.int32)
            wh = plsc.load_gather(wu_m, [hv])
            bh = plsc.load_gather(bu_m, [hv])
            ht = h // 8
            hi = lax.rem(h, 8)
            for g in range(GP):
                ubuf[ht, hi, 0, pl.ds(g * L, L)] = svs[g] * wh + bh

    def uemb_build():
        @plsc.parallel_loop(0, H, unroll=4)
        def _(h):
            hv = jnp.full((L,), h, jnp.int32)
            ht = h // 8
            hi = lax.rem(h, 8)
            for g in range(GP):
                v = plsc.load_gather(rwa, [bvecs[g], hv])
                ubuf[ht, hi, 1, pl.ds(g * L, L)] = v

    fetch_io(0, 0)

    def pair(p, carry):
        for d in range(2):
            u = 2 * p + d
            t, bt = unit_tb(u)
            gid = u * NW + wid

            io_wait(d)

            @pl.when(u + 1 <= UPW)
            def _():
                fetch_io(u + 1, 1 - d)

            @pl.when(jnp.logical_and(u < UPW, gid < UNITS))
            def _():
                fire(d, 0)
                fire(d, 1)

                @pl.when(u >= 1)
                def _():
                    uwait()

                ulin_build(d)

                @pl.when(u >= 1)
                def _():
                    ring_wait(0)

                lin_plane(d, 1, wk_m, bk_m, 0, kno_o.at[t, 0, :, bt, :, :])

                @pl.when(u >= 1)
                def _():
                    ring_wait(1)

                lin_plane(d, 2, wk_m, bk_m, 1, kno_o.at[t, 1, :, bt, :, :])
                ring_wait(0)
                lin_plane(d, 0, wo_m, bo_m, 0, obs_o.at[t, :, :, bt, :])

                gather_wait(d, 0)
                uemb_build()
                pltpu.async_copy(ubuf, unk_o.at[t, :, :, bt, :, :], usem)
                fire(d, 2)
                gather_wait(d, 1)
                ring_wait(1)
                emb_plane(1, 1, kno_o.at[t, 2, :, bt, :, :])
                fire(d, 3)
                gather_wait(d, 2)
                ring_wait(0)
                emb_plane(2, 0, kno_o.at[t, 3, :, bt, :, :])
                gather_wait(d, 3)
                ring_wait(1)
                emb_plane(3, 1, kno_o.at[t, 4, :, bt, :, :])

            @pl.when(u == UPW)
            def _():
                fire(d, 0)
                gather_wait(d, 0)

                @pl.when(wid < NBT)
                def _():
                    sbt = lax.rem(wid, NBT)
                    ring_wait(0)
                    lin_plane(d, 3, ws_m, bs_m, 0, st_o.at[0, :, sbt, :, :])
                    ring_wait(1)
                    emb_plane(0, 1, st_o.at[1, :, sbt, :, :])
        return carry

    lax.fori_loop(0, (UPW + 2) // 2, pair, 0)
    ring_wait(0)
    ring_wait(1)
    uwait()


@jax.jit
def _run(idx_t, reg_t, e0, e1, e2, e3, ws, bs, wo, bo, wu, bu, wk, bk):
    mesh = plsc.VectorSubcoreMesh(core_axis_name="c", subcore_axis_name="s")
    f = pl.kernel(
        _body, mesh=mesh,
        compiler_params=pltpu.CompilerParams(
            needs_layout_passes=False, use_tc_tiling_on_sc=False),
        out_type=[
            jax.ShapeDtypeStruct((T, HT, HI, NBT, 2, BB), jnp.float32),
            jax.ShapeDtypeStruct((T, 5, HT, NBT, HI, BB), jnp.float32),
            jax.ShapeDtypeStruct((T, HT, HI, NBT, BB), jnp.float32),
            jax.ShapeDtypeStruct((2, HT, NBT, HI, BB), jnp.float32),
        ],
        scratch_types=(
            [pltpu.VMEM((2, 4, BB), jnp.int32),
             pltpu.VMEM((2, 4, BB), jnp.float32),
             pltpu.VMEM((BB, H), jnp.float32),
             pltpu.VMEM((BB, H), jnp.float32),
             pltpu.VMEM((HT, HI, BB), jnp.float32),
             pltpu.VMEM((HT, HI, BB), jnp.float32),
             pltpu.VMEM((HT, HI, 2, BB), jnp.float32)]
            + [pltpu.VMEM((H,), jnp.float32)] * 8
            + [pltpu.SemaphoreType.DMA] * 7
        ),
    )
    return f(idx_t, reg_t, e0, e1, e2, e3, ws, bs, wo, bo, wu, bu, wk, bk)


def kernel(all_inputs, emb_0, emb_1, emb_2, emb_3, W_static, b_static,
           W_obs, b_obs, W_unknown, b_unknown, W_known, b_known):
    ai_t = all_inputs.transpose(2, 1, 0)
    idx_t = ai_t[4:].astype(jnp.int32).reshape(4, T * B)
    reg_t = ai_t[:4].reshape(4, T * B)
    unk_p, kno_p, obs_p, st_p = _run(
        idx_t, reg_t, emb_0, emb_1, emb_2, emb_3,
        W_static.reshape(H), b_static, W_obs.reshape(H), b_obs,
        W_unknown.reshape(H), b_unknown, W_known.reshape(H), b_known)
    unk = unk_p.transpose(3, 5, 0, 1, 2, 4).reshape(B, T, H, 2)
    kno = kno_p.transpose(3, 5, 0, 2, 4, 1).reshape(B, T, H, 5)
    obs = obs_p.transpose(3, 4, 0, 1, 2).reshape(B, T, H, 1)
    st = st_p.transpose(2, 4, 0, 1, 3).reshape(B, 2, H)
    return (unk, kno, obs, st)
